# baseline (speedup 1.0000x reference)
import jax
import jax.numpy as jnp
from jax.experimental import pallas as pl
from jax.experimental.pallas import tpu as pltpu

_LANE = 128


def _lane_partial_sums_f32(xt):
    n = xt.shape[-1] // _LANE
    accs = []
    for k in range(n):
        c = xt[..., k * _LANE:(k + 1) * _LANE]
        if c.dtype != jnp.float32:
            c = c.astype(jnp.float32)
        if len(accs) < 8:
            accs.append(c)
        else:
            accs[k % 8] = accs[k % 8] + c
    out = accs[0]
    for a in accs[1:]:
        out = out + a
    return out


def kernel(x, w1, b1, w2, b2):
    B, C, D, H, W = x.shape
    hid = w1.shape[0]
    S = D * H * W

    x_flat = x.reshape(B, C, S)
    b1c = b1.reshape(hid, 1)
    b2c = b2.reshape(C, 1)
    inv_s = 1.0 / float(S)

    nfull = (S // _LANE) * _LANE

    def se_kernel(x_ref, w1_ref, b1_ref, w2_ref, b2_ref, o_ref):
        xt = x_ref[0]
        if nfull:
            acc = _lane_partial_sums_f32(xt[:, :nfull])
            pooled = jnp.sum(acc, axis=-1, keepdims=True)
        else:
            pooled = jnp.zeros((C, 1), jnp.float32)
        if S > nfull:
            rem = xt[:, nfull:].astype(jnp.float32)
            pooled = pooled + jnp.sum(rem, axis=-1, keepdims=True)
        pooled = pooled * inv_s
        h = jnp.dot(w1_ref[...], pooled,
                    preferred_element_type=jnp.float32) + b1_ref[...]
        h = jnp.where(h >= 0.0, h, 0.01 * h)
        y = jnp.dot(w2_ref[...], h,
                    preferred_element_type=jnp.float32) + b2_ref[...]
        g = jax.nn.sigmoid(y)
        if g.dtype != xt.dtype:
            g = g.astype(xt.dtype)
        o_ref[0] = (xt * g).astype(o_ref.dtype)

    out_flat = pl.pallas_call(
        se_kernel,
        out_shape=jax.ShapeDtypeStruct((B, C, S), x.dtype),
        grid=(B,),
        in_specs=[
            pl.BlockSpec((1, C, S), lambda b: (b, 0, 0)),
            pl.BlockSpec((hid, C), lambda b: (0, 0)),
            pl.BlockSpec((hid, 1), lambda b: (0, 0)),
            pl.BlockSpec((C, hid), lambda b: (0, 0)),
            pl.BlockSpec((C, 1), lambda b: (0, 0)),
        ],
        out_specs=pl.BlockSpec((1, C, S), lambda b: (b, 0, 0)),
        compiler_params=pltpu.CompilerParams(
            dimension_semantics=("parallel",),
            vmem_limit_bytes=64 * 1024 * 1024),
    )(x_flat, w1, b1c, w2, b2c)

    return out_flat.reshape(B, C, D, H, W)


# speedup vs baseline: 3.3345x; 3.3345x over previous
import jax
import jax.numpy as jnp
from jax.experimental import pallas as pl
from jax.experimental.pallas import tpu as pltpu


def _sublane_mean_f32(xt, inv_s):
    S = xt.shape[0]
    ch = next((c for c in (512, 256, 128, 64, 32, 16, 8) if S % c == 0), None)
    if ch is None:
        pooled = jnp.sum(xt.astype(jnp.float32), axis=0, keepdims=True)
        return pooled * inv_s
    accs = []
    for k in range(S // ch):
        c = xt[k * ch:(k + 1) * ch, :]
        if c.dtype != jnp.float32:
            c = c.astype(jnp.float32)
        if len(accs) < 8:
            accs.append(c)
        else:
            accs[k % 8] = accs[k % 8] + c
    acc = accs[0]
    for a in accs[1:]:
        acc = acc + a
    return jnp.sum(acc, axis=0, keepdims=True) * inv_s


def kernel(x, w1, b1, w2, b2):
    B, C, D, H, W = x.shape
    hid = w1.shape[0]
    S = D * H * W
    inv_s = 1.0 / float(S)

    x_bsc = jnp.transpose(x, (0, 2, 3, 4, 1)).reshape(B, S, C)
    b1r = b1.reshape(1, hid)
    b2r = b2.reshape(1, C)

    contract_last = (((1,), (1,)), ((), ()))

    def se_kernel(x_ref, w1_ref, b1_ref, w2_ref, b2_ref, o_ref):
        xt = x_ref[0]
        pooled = _sublane_mean_f32(xt, inv_s)
        h = jax.lax.dot_general(pooled, w1_ref[...], contract_last,
                                preferred_element_type=jnp.float32)
        h = h + b1_ref[...]
        h = jnp.where(h >= 0.0, h, 0.01 * h)
        y = jax.lax.dot_general(h, w2_ref[...], contract_last,
                                preferred_element_type=jnp.float32)
        y = y + b2_ref[...]
        g = jax.nn.sigmoid(y)
        if g.dtype != xt.dtype:
            g = g.astype(xt.dtype)
        o_ref[0] = (xt * g).astype(o_ref.dtype)

    out_bsc = pl.pallas_call(
        se_kernel,
        out_shape=jax.ShapeDtypeStruct((B, S, C), x.dtype),
        grid=(B,),
        in_specs=[
            pl.BlockSpec((1, S, C), lambda b: (b, 0, 0)),
            pl.BlockSpec((hid, C), lambda b: (0, 0)),
            pl.BlockSpec((1, hid), lambda b: (0, 0)),
            pl.BlockSpec((C, hid), lambda b: (0, 0)),
            pl.BlockSpec((1, C), lambda b: (0, 0)),
        ],
        out_specs=pl.BlockSpec((1, S, C), lambda b: (b, 0, 0)),
        compiler_params=pltpu.CompilerParams(
            dimension_semantics=("parallel",),
            vmem_limit_bytes=64 * 1024 * 1024),
    )(x_bsc, w1, b1r, w2, b2r)

    return jnp.transpose(out_bsc.reshape(B, D, H, W, C), (0, 4, 1, 2, 3))


# speedup vs baseline: 3.7549x; 1.1261x over previous
import jax
import jax.numpy as jnp
from jax.experimental import pallas as pl
from jax.experimental.pallas import tpu as pltpu


def _sublane_mean_f32(xt, inv_s):
    S = xt.shape[0]
    ch = next((c for c in (512, 256, 128, 64, 32, 16, 8) if S % c == 0), None)
    if ch is None:
        pooled = jnp.sum(xt.astype(jnp.float32), axis=0, keepdims=True)
        return pooled * inv_s
    accs = []
    for k in range(S // ch):
        c = xt[k * ch:(k + 1) * ch, :]
        if c.dtype != jnp.float32:
            c = c.astype(jnp.float32)
        if len(accs) < 8:
            accs.append(c)
        else:
            accs[k % 8] = accs[k % 8] + c
    acc = accs[0]
    for a in accs[1:]:
        acc = acc + a
    return jnp.sum(acc, axis=0, keepdims=True) * inv_s


def kernel(x, w1, b1, w2, b2):
    B, C, D, H, W = x.shape
    hid = w1.shape[0]
    S = D * H * W
    inv_s = 1.0 / float(S)

    x_bsc = jnp.transpose(x, (0, 2, 3, 4, 1)).reshape(B, S, C)
    b1r = b1.reshape(1, hid)
    b2r = b2.reshape(1, C)

    contract_last = (((1,), (1,)), ((), ()))
    Bt = 2 if B % 2 == 0 else 1

    def se_kernel(x_ref, w1_ref, b1_ref, w2_ref, b2_ref, o_ref):
        xt = x_ref[...]
        rows = [_sublane_mean_f32(xt[i], inv_s) for i in range(Bt)]
        pooled = rows[0] if Bt == 1 else jnp.concatenate(rows, axis=0)
        h = jax.lax.dot_general(pooled, w1_ref[...], contract_last,
                                preferred_element_type=jnp.float32)
        h = h + b1_ref[...]
        h = jnp.where(h >= 0.0, h, 0.01 * h)
        y = jax.lax.dot_general(h, w2_ref[...], contract_last,
                                preferred_element_type=jnp.float32)
        y = y + b2_ref[...]
        g = jax.nn.sigmoid(y)
        if g.dtype != xt.dtype:
            g = g.astype(xt.dtype)
        o_ref[...] = (xt * g[:, None, :]).astype(o_ref.dtype)

    out_bsc = pl.pallas_call(
        se_kernel,
        out_shape=jax.ShapeDtypeStruct((B, S, C), x.dtype),
        grid=(B // Bt,),
        in_specs=[
            pl.BlockSpec((Bt, S, C), lambda b: (b, 0, 0)),
            pl.BlockSpec((hid, C), lambda b: (0, 0)),
            pl.BlockSpec((1, hid), lambda b: (0, 0)),
            pl.BlockSpec((C, hid), lambda b: (0, 0)),
            pl.BlockSpec((1, C), lambda b: (0, 0)),
        ],
        out_specs=pl.BlockSpec((Bt, S, C), lambda b: (b, 0, 0)),
        compiler_params=pltpu.CompilerParams(
            dimension_semantics=("parallel",),
            vmem_limit_bytes=64 * 1024 * 1024),
    )(x_bsc, w1, b1r, w2, b2r)

    return jnp.transpose(out_bsc.reshape(B, D, H, W, C), (0, 4, 1, 2, 3))


# speedup vs baseline: 3.9970x; 1.0645x over previous
import jax
import jax.numpy as jnp
from jax.experimental import pallas as pl
from jax.experimental.pallas import tpu as pltpu


def _sublane_mean_f32(xt, inv_s):
    S = xt.shape[0]
    ch = next((c for c in (512, 256, 128, 64, 32, 16, 8) if S % c == 0), None)
    if ch is None:
        pooled = jnp.sum(xt.astype(jnp.float32), axis=0, keepdims=True)
        return pooled * inv_s
    accs = []
    for k in range(S // ch):
        c = xt[k * ch:(k + 1) * ch, :]
        if c.dtype != jnp.float32:
            c = c.astype(jnp.float32)
        if len(accs) < 8:
            accs.append(c)
        else:
            accs[k % 8] = accs[k % 8] + c
    acc = accs[0]
    for a in accs[1:]:
        acc = acc + a
    return jnp.sum(acc, axis=0, keepdims=True) * inv_s


def kernel(x, w1, b1, w2, b2):
    B, C, D, H, W = x.shape
    hid = w1.shape[0]
    S = D * H * W
    inv_s = 1.0 / float(S)

    x_bsc = jnp.transpose(x, (0, 2, 3, 4, 1)).reshape(B, S, C)
    b1r = b1.reshape(1, hid)
    b2r = b2.reshape(1, C)

    contract_last = (((1,), (1,)), ((), ()))
    Bt = 4 if B % 4 == 0 else (2 if B % 2 == 0 else 1)

    def se_kernel(x_ref, w1_ref, b1_ref, w2_ref, b2_ref, o_ref):
        xt = x_ref[...]
        rows = [_sublane_mean_f32(xt[i], inv_s) for i in range(Bt)]
        pooled = rows[0] if Bt == 1 else jnp.concatenate(rows, axis=0)
        h = jax.lax.dot_general(pooled, w1_ref[...], contract_last,
                                preferred_element_type=jnp.float32)
        h = h + b1_ref[...]
        h = jnp.where(h >= 0.0, h, 0.01 * h)
        y = jax.lax.dot_general(h, w2_ref[...], contract_last,
                                preferred_element_type=jnp.float32)
        y = y + b2_ref[...]
        g = jax.nn.sigmoid(y)
        if g.dtype != xt.dtype:
            g = g.astype(xt.dtype)
        o_ref[...] = (xt * g[:, None, :]).astype(o_ref.dtype)

    out_bsc = pl.pallas_call(
        se_kernel,
        out_shape=jax.ShapeDtypeStruct((B, S, C), x.dtype),
        grid=(B // Bt,),
        in_specs=[
            pl.BlockSpec((Bt, S, C), lambda b: (b, 0, 0)),
            pl.BlockSpec((hid, C), lambda b: (0, 0)),
            pl.BlockSpec((1, hid), lambda b: (0, 0)),
            pl.BlockSpec((C, hid), lambda b: (0, 0)),
            pl.BlockSpec((1, C), lambda b: (0, 0)),
        ],
        out_specs=pl.BlockSpec((Bt, S, C), lambda b: (b, 0, 0)),
        compiler_params=pltpu.CompilerParams(
            dimension_semantics=("parallel",),
            vmem_limit_bytes=64 * 1024 * 1024),
    )(x_bsc, w1, b1r, w2, b2r)

    return jnp.transpose(out_bsc.reshape(B, D, H, W, C), (0, 4, 1, 2, 3))
